# baseline (speedup 1.0000x reference)
import functools

import jax
import jax.numpy as jnp
from jax import lax
from jax.experimental import pallas as pl
from jax.experimental.pallas import tpu as pltpu
from jax.experimental.pallas import tpu_sc as plsc

NC = 2
NS = 16
NW = NC * NS
CHUNK = 128
NG = 4


@functools.partial(jax.jit, static_argnums=(1,))
def _sc_gather(args, dims):
    B, L, D = dims
    rows_per_w = B // NW
    n_groups = rows_per_w // NG
    chunks = [(s, min(CHUNK, L - s)) for s in range(0, L, CHUNK)]
    mesh = plsc.VectorSubcoreMesh(core_axis_name="c", subcore_axis_name="s")

    bufs = [pltpu.VMEM((NG, size, D), jnp.float32) for _, size in chunks]
    n_dma = NG * len(chunks)

    @functools.partial(
        pl.kernel,
        mesh=mesh,
        out_type=jax.ShapeDtypeStruct((B, L, D), jnp.float32),
        scratch_types=[pltpu.VMEM((rows_per_w, L), jnp.int32)] + bufs
        + [pltpu.SemaphoreType.DMA] * n_dma,
        compiler_params=pltpu.CompilerParams(use_tc_tiling_on_sc=False),
    )
    def k(x_hbm, table_hbm, out_hbm, idx_v, *rest):
        rows_v = rest[:len(chunks)]
        sems = rest[len(chunks):]
        wid = lax.axis_index("s") * NC + lax.axis_index("c")
        row0 = wid * rows_per_w
        pltpu.sync_copy(x_hbm.at[pl.ds(row0, rows_per_w), :], idx_v)

        def body(g, carry):
            r0 = g * NG
            handles = []
            for j in range(NG):
                for ci, (s, size) in enumerate(chunks):
                    handles.append(pltpu.async_copy(
                        table_hbm.at[idx_v.at[r0 + j, pl.ds(s, size)]],
                        rows_v[ci].at[j],
                        sems[j * len(chunks) + ci]))
            h = 0
            for j in range(NG):
                b = row0 + r0 + j
                for ci, (s, size) in enumerate(chunks):
                    handles[h].wait()
                    h += 1
                    pltpu.sync_copy(rows_v[ci].at[j],
                                    out_hbm.at[b, pl.ds(s, size), :])
            return carry

        lax.fori_loop(0, n_groups, body, 0)

    x, table = args
    return k(x, table)


def kernel(x, table):
    B, L = x.shape
    V, D = table.shape
    return _sc_gather((x.astype(jnp.int32), table), (B, L, D))
